# Initial kernel scaffold; baseline (speedup 1.0000x reference)
import jax
import jax.numpy as jnp
from jax.experimental import pallas as pl


def kernel(queries, keys, values, alpha, beta):
    raise NotImplementedError("write your pallas kernel here")


# speedup vs baseline: 4.1091x; 4.1091x over previous
import functools
import jax
import jax.numpy as jnp
from jax import lax
from jax.experimental import pallas as pl
from jax.experimental.pallas import tpu as pltpu
from jax.experimental.pallas import tpu_sc as plsc

NH = 2
BS = 32
SEQ = 4096
DIM = 64
BLK = 128
NBLK = SEQ // BLK
NROWS = NH * BS * SEQ
NTAB = BS * SEQ
NC, NS = 2, 16
NW = NC * NS
CHUNK = 128
ROWS_PER_W = NROWS // NW
CHUNKS_PER_W = ROWS_PER_W // CHUNK
NBUF = 2
EW = 128


def _worker_id():
    return lax.axis_index("s") * NC + lax.axis_index("c")


def _sc_gather_body(qt, kt, vt, qidx, kidx, sq, sk, sv,
                    qi_v, ki_v, buf_a, buf_b, sems_a, sems_b):
    w = _worker_id()
    pltpu.sync_copy(qidx.at[w], qi_v)
    pltpu.sync_copy(kidx.at[w], ki_v)
    base = w * ROWS_PER_W

    def q_group(it, _):
        jo = it * NBUF
        cps = []
        for b in range(NBUF):
            cps.append(pltpu.async_copy(
                qt.at[qi_v.at[jo + b]], buf_a.at[b], sems_a.at[b]))
        for b in range(NBUF):
            cps[b].wait()
            off = pl.multiple_of(base + (jo + b) * CHUNK, CHUNK)
            pltpu.sync_copy(buf_a.at[b], sq.at[pl.ds(off, CHUNK)])
        return 0

    lax.fori_loop(0, CHUNKS_PER_W // NBUF, q_group, 0)

    def kv_group(it, _):
        jo = it * NBUF
        kcps, vcps = [], []
        for b in range(NBUF):
            kcps.append(pltpu.async_copy(
                kt.at[ki_v.at[jo + b]], buf_a.at[b], sems_a.at[b]))
            vcps.append(pltpu.async_copy(
                vt.at[ki_v.at[jo + b]], buf_b.at[b], sems_b.at[b]))
        for b in range(NBUF):
            off = pl.multiple_of(base + (jo + b) * CHUNK, CHUNK)
            kcps[b].wait()
            pltpu.sync_copy(buf_a.at[b], sk.at[pl.ds(off, CHUNK)])
            vcps[b].wait()
            pltpu.sync_copy(buf_b.at[b], sv.at[pl.ds(off, CHUNK)])
        return 0

    lax.fori_loop(0, CHUNKS_PER_W // NBUF, kv_group, 0)


def _sc_mesh():
    return plsc.VectorSubcoreMesh(
        core_axis_name="c", subcore_axis_name="s",
        num_cores=NC, num_subcores=NS)


@functools.lru_cache(maxsize=1)
def _make_sc_gather():
    return functools.partial(
        pl.kernel,
        out_type=(
            jax.ShapeDtypeStruct((NROWS, EW), jnp.float32),
            jax.ShapeDtypeStruct((NROWS, EW), jnp.float32),
            jax.ShapeDtypeStruct((NROWS, EW), jnp.float32),
        ),
        mesh=_sc_mesh(),
        scratch_types=[
            pltpu.VMEM((CHUNKS_PER_W, CHUNK), jnp.int32),
            pltpu.VMEM((CHUNKS_PER_W, CHUNK), jnp.int32),
            pltpu.VMEM((NBUF, CHUNK, EW), jnp.float32),
            pltpu.VMEM((NBUF, CHUNK, EW), jnp.float32),
            pltpu.SemaphoreType.DMA((NBUF,)),
            pltpu.SemaphoreType.DMA((NBUF,)),
        ],
    )(_sc_gather_body)


def _sc_gather(*args):
    return _make_sc_gather()(*args)


def _attn_body(sq_ref, sk_ref, sv_ref, cq_ref, ck_ref, cv_ref,
               ext_ref, cbo_ref):
    q = sq_ref[0]
    k = sk_ref[0]
    v = sv_ref[0]
    cq = cq_ref[0]
    ck = ck_ref[0]
    cv = cv_ref[0]
    dims = (((1,), (1,)), ((), ()))
    s = lax.dot_general(q, k, dims, preferred_element_type=jnp.float32)
    sc = lax.dot_general(q, ck, dims, preferred_element_type=jnp.float32)
    m = jnp.maximum(jnp.max(s, axis=1, keepdims=True), sc)
    e = jnp.exp(s - m)
    ec = jnp.exp(sc - m)
    den = jnp.sum(e, axis=1, keepdims=True) + ec
    bo = (jnp.dot(e, v, preferred_element_type=jnp.float32) + ec * cv) / den
    lse = m + jnp.log(den)
    colid = lax.broadcasted_iota(jnp.int32, (BLK, EW), 1)
    ext_ref[0] = jnp.where(colid == DIM, lse, bo)
    t = lax.dot_general(cq, k, dims, preferred_element_type=jnp.float32)
    tc = lax.dot_general(cq, ck, dims, preferred_element_type=jnp.float32)
    m2 = jnp.maximum(jnp.max(t, axis=1, keepdims=True), tc)
    e2 = jnp.exp(t - m2)
    e2c = jnp.exp(tc - m2)
    den2 = jnp.sum(e2, axis=1, keepdims=True) + e2c
    cbo_ref[0] = (jnp.dot(e2, v, preferred_element_type=jnp.float32)
                  + e2c * cv) / den2


def _attention(sq, sk, sv, cls_q, cls_k, cls_v, interpret=False):
    nblk_total = NH * BS * NBLK
    blk_spec = pl.BlockSpec((1, BLK, EW), lambda i: (i, 0, 0))
    cls_spec = pl.BlockSpec((1, 1, EW), lambda i: ((i // NBLK) % BS, 0, 0))
    return pl.pallas_call(
        _attn_body,
        grid=(nblk_total,),
        in_specs=[blk_spec, blk_spec, blk_spec, cls_spec, cls_spec, cls_spec],
        out_specs=[
            pl.BlockSpec((1, BLK, EW), lambda i: (i, 0, 0)),
            pl.BlockSpec((1, 1, EW), lambda i: (i, 0, 0)),
        ],
        out_shape=[
            jax.ShapeDtypeStruct((nblk_total, BLK, EW), jnp.float32),
            jax.ShapeDtypeStruct((nblk_total, 1, EW), jnp.float32),
        ],
        interpret=interpret,
    )(sq, sk, sv, cls_q, cls_k, cls_v)


def _sc_merge_body(ext, gidx, out, i0_v, i1_v, r0, r1, ob, sem):
    w = _worker_id()

    def chunk(c, _):
        pltpu.sync_copy(gidx.at[0, w, c], i0_v)
        pltpu.sync_copy(gidx.at[1, w, c], i1_v)
        cp0 = pltpu.async_copy(ext.at[i0_v], r0, sem)
        cp1 = pltpu.async_copy(ext.at[i1_v], r1, sem)
        cp0.wait()
        cp1.wait()
        for r in range(BLK):
            la = r0[r, pl.ds(DIM, 16)]
            lb = r1[r, pl.ds(DIM, 16)]
            wv = 1.0 / (1.0 + jnp.exp(lb - la))
            ws = wv[0]
            for cg in range(4):
                a = r0[r, pl.ds(cg * 16, 16)]
                b = r1[r, pl.ds(cg * 16, 16)]
                ob[r, pl.ds(cg * 16, 16)] = b + ws * (a - b)
        off = pl.multiple_of((w * SEQ + c * BLK), BLK)
        pltpu.sync_copy(ob, out.at[pl.ds(off, BLK)])
        return 0

    lax.fori_loop(0, NBLK, chunk, 0)


@functools.lru_cache(maxsize=1)
def _make_sc_merge():
    return functools.partial(
        pl.kernel,
        out_type=jax.ShapeDtypeStruct((NTAB, EW), jnp.float32),
        mesh=_sc_mesh(),
        scratch_types=[
            pltpu.VMEM((BLK,), jnp.int32),
            pltpu.VMEM((BLK,), jnp.int32),
            pltpu.VMEM((BLK, EW), jnp.float32),
            pltpu.VMEM((BLK, EW), jnp.float32),
            pltpu.VMEM((BLK, EW), jnp.float32),
            pltpu.SemaphoreType.DMA,
        ],
    )(_sc_merge_body)


def _sc_merge(*args):
    return _make_sc_merge()(*args)


def kernel(queries, keys, values, alpha, beta):
    cls_q = queries[:, :1]
    cls_k = keys[:, :1]
    cls_v = values[:, :1]
    q = queries[:, 1:]
    k = keys[:, 1:]
    v = values[:, 1:]

    q_sg = lax.stop_gradient(q)
    k_sg = lax.stop_gradient(k)
    q_norm = jnp.linalg.norm(q_sg, axis=-1, keepdims=True)
    k_norm = jnp.linalg.norm(k_sg, axis=-1, keepdims=True)
    MQ = jnp.max(q_norm, axis=1, keepdims=True)
    MK = jnp.max(k_norm, axis=1, keepdims=True)
    q_ext = jnp.sqrt(jnp.maximum(MQ ** 2 + MK ** 2 - q_norm ** 2, 0.0))
    k_ext = jnp.sqrt(jnp.maximum(MQ ** 2 + MK ** 2 - k_norm ** 2, 0.0))
    Queries = jnp.concatenate([q_sg, q_ext, jnp.zeros_like(q_ext)], axis=-1)
    Keys = jnp.concatenate([k_sg, jnp.zeros_like(k_ext), k_ext], axis=-1)
    a = lax.stop_gradient(alpha)
    b = lax.stop_gradient(beta)
    q_hashed = jnp.transpose(Queries @ a + b, (2, 0, 1))
    k_hashed = jnp.transpose(Keys @ a + b, (2, 0, 1))
    q_positions = jnp.argsort(q_hashed, axis=-1).astype(jnp.int32)
    k_positions = jnp.argsort(k_hashed, axis=-1).astype(jnp.int32)

    boff = (jnp.arange(BS, dtype=jnp.int32) * SEQ)[None, :, None]
    q_gidx = (q_positions + boff).reshape(NW, CHUNKS_PER_W, CHUNK)
    k_gidx = (k_positions + boff).reshape(NW, CHUNKS_PER_W, CHUNK)

    pad = ((0, 0), (0, EW - DIM))
    qt = jnp.pad(q.reshape(NTAB, DIM), pad)
    kt = jnp.pad(k.reshape(NTAB, DIM), pad)
    vt = jnp.pad(v.reshape(NTAB, DIM), pad)
    sq, sk, sv = _sc_gather(qt, kt, vt, q_gidx, k_gidx)

    pad3 = ((0, 0), (0, 0), (0, EW - DIM))
    nblk_total = NH * BS * NBLK
    ext, cbo = _attention(
        sq.reshape(nblk_total, BLK, EW),
        sk.reshape(nblk_total, BLK, EW),
        sv.reshape(nblk_total, BLK, EW),
        jnp.pad(cls_q, pad3), jnp.pad(cls_k, pad3), jnp.pad(cls_v, pad3))

    flatpos = (jnp.arange(NH * BS, dtype=jnp.int32) * SEQ)[:, None] \
        + jnp.arange(SEQ, dtype=jnp.int32)[None, :]
    gidx = jnp.put_along_axis(
        jnp.zeros((NH, BS, SEQ), jnp.int32), q_positions,
        flatpos.reshape(NH, BS, SEQ), axis=-1, inplace=False)
    gidx = gidx.reshape(NH, BS, NBLK, CHUNK)

    merged = _sc_merge(ext.reshape(NROWS, EW), gidx)
    out_body = merged[:, :DIM].reshape(BS, SEQ, DIM)
    cls_out = cbo.reshape(NH, BS, NBLK, EW)[..., :DIM].mean(
        axis=(0, 2))[:, None, :]
    return jnp.concatenate([cls_out, out_body], axis=1)


# speedup vs baseline: 6.3837x; 1.5535x over previous
import functools
import jax
import jax.numpy as jnp
from jax import lax
from jax.experimental import pallas as pl
from jax.experimental.pallas import tpu as pltpu
from jax.experimental.pallas import tpu_sc as plsc

NH = 2
BS = 32
SEQ = 4096
DIM = 64
BLK = 128
NBLK = SEQ // BLK
NROWS = NH * BS * SEQ
NTAB = BS * SEQ
NC, NS = 2, 16
NW = NC * NS
CHUNK = 128
ROWS_PER_W = NROWS // NW
CHUNKS_PER_W = ROWS_PER_W // CHUNK
NBUF = 2
EW = 128


def _worker_id():
    return lax.axis_index("s") * NC + lax.axis_index("c")


def _sc_gather_body(qt, kt, vt, qidx, kidx, rtidx, sq, sk, sv, rev,
                    qi_v, ki_v, rt_v, ramp_v, val_v, buf_a, buf_b,
                    sems_a, sems_b, sem_r):
    w = _worker_id()
    pltpu.sync_copy(qidx.at[w], qi_v)
    pltpu.sync_copy(kidx.at[w], ki_v)
    pltpu.sync_copy(rtidx.at[w], rt_v)
    base = w * ROWS_PER_W

    lane = lax.iota(jnp.int32, 16)
    for s in range(8):
        ramp_v[pl.ds(s * 16, 16)] = lane + (s * 16)

    def q_group(it, _):
        jo = it * NBUF
        cps = []
        for b in range(NBUF):
            cps.append(pltpu.async_copy(
                qt.at[qi_v.at[jo + b]], buf_a.at[b], sems_a.at[b]))
        rcps = []
        for b in range(NBUF):
            cbase = base + (jo + b) * CHUNK
            for s in range(8):
                val_v[b, pl.ds(s * 16, 16)] = \
                    ramp_v[pl.ds(s * 16, 16)] + cbase
            rcps.append(pltpu.async_copy(
                val_v.at[b], rev.at[rt_v.at[jo + b]], sem_r))
        for b in range(NBUF):
            cps[b].wait()
            off = pl.multiple_of(base + (jo + b) * CHUNK, CHUNK)
            pltpu.sync_copy(buf_a.at[b], sq.at[pl.ds(off, CHUNK)])
            rcps[b].wait()
        return 0

    lax.fori_loop(0, CHUNKS_PER_W // NBUF, q_group, 0)

    def kv_group(it, _):
        jo = it * NBUF
        kcps, vcps = [], []
        for b in range(NBUF):
            kcps.append(pltpu.async_copy(
                kt.at[ki_v.at[jo + b]], buf_a.at[b], sems_a.at[b]))
            vcps.append(pltpu.async_copy(
                vt.at[ki_v.at[jo + b]], buf_b.at[b], sems_b.at[b]))
        for b in range(NBUF):
            off = pl.multiple_of(base + (jo + b) * CHUNK, CHUNK)
            kcps[b].wait()
            pltpu.sync_copy(buf_a.at[b], sk.at[pl.ds(off, CHUNK)])
            vcps[b].wait()
            pltpu.sync_copy(buf_b.at[b], sv.at[pl.ds(off, CHUNK)])
        return 0

    lax.fori_loop(0, CHUNKS_PER_W // NBUF, kv_group, 0)


def _sc_mesh():
    return plsc.VectorSubcoreMesh(
        core_axis_name="c", subcore_axis_name="s",
        num_cores=NC, num_subcores=NS)


@functools.lru_cache(maxsize=1)
def _make_sc_gather():
    return functools.partial(
        pl.kernel,
        out_type=(
            jax.ShapeDtypeStruct((NROWS, EW), jnp.float32),
            jax.ShapeDtypeStruct((NROWS, EW), jnp.float32),
            jax.ShapeDtypeStruct((NROWS, EW), jnp.float32),
            jax.ShapeDtypeStruct((NROWS,), jnp.int32),
        ),
        mesh=_sc_mesh(),
        scratch_types=[
            pltpu.VMEM((CHUNKS_PER_W, CHUNK), jnp.int32),
            pltpu.VMEM((CHUNKS_PER_W, CHUNK), jnp.int32),
            pltpu.VMEM((CHUNKS_PER_W, CHUNK), jnp.int32),
            pltpu.VMEM((CHUNK,), jnp.int32),
            pltpu.VMEM((NBUF, CHUNK), jnp.int32),
            pltpu.VMEM((NBUF, CHUNK, EW), jnp.float32),
            pltpu.VMEM((NBUF, CHUNK, EW), jnp.float32),
            pltpu.SemaphoreType.DMA((NBUF,)),
            pltpu.SemaphoreType.DMA((NBUF,)),
            pltpu.SemaphoreType.DMA,
        ],
    )(_sc_gather_body)


def _sc_gather(*args):
    return _make_sc_gather()(*args)


GA = 8


def _attn_body(sq_ref, sk_ref, sv_ref, cq_ref, ck_ref, cv_ref,
               ext_ref, cbo_ref):
    dt = (((1,), (1,)), ((), ()))
    dk = (((0,), (0,)), ((), ()))
    cq = cq_ref[0]
    ck = ck_ref[0]
    cv = cv_ref[0]
    colid = lax.broadcasted_iota(jnp.int32, (BLK, EW), 1)
    qs = [sq_ref[g] for g in range(GA)]
    ks = [sk_ref[g] for g in range(GA)]
    vs = [sv_ref[g] for g in range(GA)]
    sts = [lax.dot_general(ks[g], qs[g], dt,
                           preferred_element_type=jnp.float32)
           for g in range(GA)]
    scrs = [lax.dot_general(ck, qs[g], dt,
                            preferred_element_type=jnp.float32)
            for g in range(GA)]
    tcols = [lax.dot_general(ks[g], cq, dt,
                             preferred_element_type=jnp.float32)
             for g in range(GA)]
    tcs = lax.dot_general(ck, cq, dt, preferred_element_type=jnp.float32)
    ets, ecs, dens, lses = [], [], [], []
    for g in range(GA):
        mr = jnp.maximum(jnp.max(sts[g], axis=0, keepdims=True), scrs[g])
        ets.append(jnp.exp(sts[g] - mr))
        ecs.append(jnp.exp(scrs[g] - mr))
        denr = jnp.sum(ets[g], axis=0, keepdims=True) + ecs[g]
        dens.append(denr)
        lses.append(mr + jnp.log(denr))
    pvs = [lax.dot_general(ets[g], vs[g], dk,
                           preferred_element_type=jnp.float32)
           for g in range(GA)]
    for g in range(GA):
        ec_c = jnp.transpose(ecs[g])
        den_c = jnp.transpose(dens[g])
        lse_c = jnp.transpose(lses[g])
        bo = (pvs[g] + ec_c * cv) / den_c
        ext_ref[g] = jnp.where(colid == DIM, lse_c, bo)
    e2s = []
    for g in range(GA):
        m2 = jnp.maximum(jnp.max(tcols[g], axis=0, keepdims=True), tcs)
        e2s.append((jnp.exp(tcols[g] - m2), jnp.exp(tcs - m2)))
    for g in range(GA):
        e2, e2c = e2s[g]
        den2 = jnp.sum(e2, axis=0, keepdims=True) + e2c
        cbo_ref[g] = (lax.dot_general(
            e2, vs[g], dk, preferred_element_type=jnp.float32)
            + e2c * cv) / den2


def _attention(sq, sk, sv, cls_q, cls_k, cls_v, interpret=False):
    nblk_total = NH * BS * NBLK
    blk_spec = pl.BlockSpec((GA, BLK, EW), lambda i: (i, 0, 0))
    cls_spec = pl.BlockSpec(
        (1, 1, EW), lambda i: ((i * GA) // NBLK % BS, 0, 0))
    return pl.pallas_call(
        _attn_body,
        grid=(nblk_total // GA,),
        in_specs=[blk_spec, blk_spec, blk_spec, cls_spec, cls_spec, cls_spec],
        out_specs=[
            pl.BlockSpec((GA, BLK, EW), lambda i: (i, 0, 0)),
            pl.BlockSpec((GA, 1, EW), lambda i: (i, 0, 0)),
        ],
        out_shape=[
            jax.ShapeDtypeStruct((nblk_total, BLK, EW), jnp.float32),
            jax.ShapeDtypeStruct((nblk_total, 1, EW), jnp.float32),
        ],
        interpret=interpret,
    )(sq, sk, sv, cls_q, cls_k, cls_v)


def _sc_merge_body(ext, gidx, out, i0_v, i1_v, r0, r1, ob, sem):
    w = _worker_id()

    def chunk(c, _):
        pltpu.sync_copy(gidx.at[0, w, c], i0_v)
        pltpu.sync_copy(gidx.at[1, w, c], i1_v)
        cp0 = pltpu.async_copy(ext.at[i0_v], r0, sem)
        cp1 = pltpu.async_copy(ext.at[i1_v], r1, sem)
        cp0.wait()
        cp1.wait()
        for r in range(BLK):
            la = r0[r, pl.ds(DIM, 16)]
            lb = r1[r, pl.ds(DIM, 16)]
            wv = 1.0 / (1.0 + jnp.exp(lb - la))
            ws = wv[0]
            for cg in range(4):
                a = r0[r, pl.ds(cg * 16, 16)]
                b = r1[r, pl.ds(cg * 16, 16)]
                ob[r, pl.ds(cg * 16, 16)] = b + ws * (a - b)
        off = pl.multiple_of((w * SEQ + c * BLK), BLK)
        pltpu.sync_copy(ob, out.at[pl.ds(off, BLK)])
        return 0

    lax.fori_loop(0, NBLK, chunk, 0)


@functools.lru_cache(maxsize=1)
def _make_sc_merge():
    return functools.partial(
        pl.kernel,
        out_type=jax.ShapeDtypeStruct((NTAB, EW), jnp.float32),
        mesh=_sc_mesh(),
        scratch_types=[
            pltpu.VMEM((BLK,), jnp.int32),
            pltpu.VMEM((BLK,), jnp.int32),
            pltpu.VMEM((BLK, EW), jnp.float32),
            pltpu.VMEM((BLK, EW), jnp.float32),
            pltpu.VMEM((BLK, EW), jnp.float32),
            pltpu.SemaphoreType.DMA,
        ],
    )(_sc_merge_body)


def _sc_merge(*args):
    return _make_sc_merge()(*args)


def kernel(queries, keys, values, alpha, beta):
    cls_q = queries[:, :1]
    cls_k = keys[:, :1]
    cls_v = values[:, :1]
    q = queries[:, 1:]
    k = keys[:, 1:]
    v = values[:, 1:]

    q_sg = lax.stop_gradient(q)
    k_sg = lax.stop_gradient(k)
    q_norm = jnp.linalg.norm(q_sg, axis=-1, keepdims=True)
    k_norm = jnp.linalg.norm(k_sg, axis=-1, keepdims=True)
    MQ = jnp.max(q_norm, axis=1, keepdims=True)
    MK = jnp.max(k_norm, axis=1, keepdims=True)
    q_ext = jnp.sqrt(jnp.maximum(MQ ** 2 + MK ** 2 - q_norm ** 2, 0.0))
    k_ext = jnp.sqrt(jnp.maximum(MQ ** 2 + MK ** 2 - k_norm ** 2, 0.0))
    Queries = jnp.concatenate([q_sg, q_ext, jnp.zeros_like(q_ext)], axis=-1)
    Keys = jnp.concatenate([k_sg, jnp.zeros_like(k_ext), k_ext], axis=-1)
    a = lax.stop_gradient(alpha)
    b = lax.stop_gradient(beta)
    q_hashed = jnp.transpose(Queries @ a + b, (2, 0, 1))
    k_hashed = jnp.transpose(Keys @ a + b, (2, 0, 1))
    q_positions = jnp.argsort(q_hashed, axis=-1).astype(jnp.int32)
    k_positions = jnp.argsort(k_hashed, axis=-1).astype(jnp.int32)

    boff = (jnp.arange(BS, dtype=jnp.int32) * SEQ)[None, :, None]
    q_gidx = (q_positions + boff).reshape(NW, CHUNKS_PER_W, CHUNK)
    k_gidx = (k_positions + boff).reshape(NW, CHUNKS_PER_W, CHUNK)
    hboff = (jnp.arange(NH, dtype=jnp.int32) * BS)[:, None, None] * SEQ
    rtidx = (q_positions + boff + hboff).reshape(NW, CHUNKS_PER_W, CHUNK)

    pad = ((0, 0), (0, EW - DIM))
    qt = jnp.pad(q.reshape(NTAB, DIM), pad)
    kt = jnp.pad(k.reshape(NTAB, DIM), pad)
    vt = jnp.pad(v.reshape(NTAB, DIM), pad)
    sq, sk, sv, rev = _sc_gather(qt, kt, vt, q_gidx, k_gidx, rtidx)

    pad3 = ((0, 0), (0, 0), (0, EW - DIM))
    nblk_total = NH * BS * NBLK
    ext, cbo = _attention(
        sq.reshape(nblk_total, BLK, EW),
        sk.reshape(nblk_total, BLK, EW),
        sv.reshape(nblk_total, BLK, EW),
        jnp.pad(cls_q, pad3), jnp.pad(cls_k, pad3), jnp.pad(cls_v, pad3))

    gidx = rev.reshape(NH, BS, NBLK, CHUNK)

    merged = _sc_merge(ext.reshape(NROWS, EW), gidx)
    out_body = merged[:, :DIM].reshape(BS, SEQ, DIM)
    cls_out = cbo.reshape(NH, BS, NBLK, EW)[..., :DIM].mean(
        axis=(0, 2))[:, None, :]
    return jnp.concatenate([cls_out, out_body], axis=1)


# speedup vs baseline: 7.8413x; 1.2283x over previous
import functools
import jax
import jax.numpy as jnp
from jax import lax
from jax.experimental import pallas as pl
from jax.experimental.pallas import tpu as pltpu
from jax.experimental.pallas import tpu_sc as plsc

NH = 2
BS = 32
SEQ = 4096
DIM = 64
BLK = 128
NBLK = SEQ // BLK
NROWS = NH * BS * SEQ
NTAB = BS * SEQ
NC, NS = 2, 16
NW = NC * NS
CHUNK = 128
ROWS_PER_W = NROWS // NW
CHUNKS_PER_W = ROWS_PER_W // CHUNK
NBUF = 2
EW = 128


def _worker_id():
    return lax.axis_index("s") * NC + lax.axis_index("c")


def _sc_gather_body(qt, kt, vt, qidx, kidx, rtidx, sq, sk, sv, rev,
                    qi_v, ki_v, rt_v, ramp_v, rev_v, buf_a, buf_b,
                    sems_a, sems_b):
    w = _worker_id()
    pltpu.sync_copy(qidx.at[w], qi_v)
    pltpu.sync_copy(kidx.at[w], ki_v)
    pltpu.sync_copy(rtidx.at[w], rt_v)
    base = w * ROWS_PER_W

    lane = lax.iota(jnp.int32, 16)
    for s in range(8):
        ramp_v[pl.ds(s * 16, 16)] = lane + (s * 16)

    def q_group(it, _):
        jo = it * NBUF
        cps = []
        for b in range(NBUF):
            cps.append(pltpu.async_copy(
                qt.at[qi_v.at[jo + b]], buf_a.at[b], sems_a.at[b]))
        for b in range(NBUF):
            cbase = base + (jo + b) * CHUNK
            for s in range(8):
                tgt = rt_v[jo + b, pl.ds(s * 16, 16)]
                vals = ramp_v[pl.ds(s * 16, 16)] + cbase
                plsc.store_scatter(rev_v, [tgt], vals)
        for b in range(NBUF):
            cps[b].wait()
            off = pl.multiple_of(base + (jo + b) * CHUNK, CHUNK)
            pltpu.sync_copy(buf_a.at[b], sq.at[pl.ds(off, CHUNK)])
        return 0

    lax.fori_loop(0, CHUNKS_PER_W // NBUF, q_group, 0)
    pltpu.sync_copy(rev_v, rev.at[pl.ds(base, ROWS_PER_W)])

    def kv_group(it, _):
        jo = it * NBUF
        kcps, vcps = [], []
        for b in range(NBUF):
            kcps.append(pltpu.async_copy(
                kt.at[ki_v.at[jo + b]], buf_a.at[b], sems_a.at[b]))
            vcps.append(pltpu.async_copy(
                vt.at[ki_v.at[jo + b]], buf_b.at[b], sems_b.at[b]))
        for b in range(NBUF):
            off = pl.multiple_of(base + (jo + b) * CHUNK, CHUNK)
            kcps[b].wait()
            pltpu.sync_copy(buf_a.at[b], sk.at[pl.ds(off, CHUNK)])
            vcps[b].wait()
            pltpu.sync_copy(buf_b.at[b], sv.at[pl.ds(off, CHUNK)])
        return 0

    lax.fori_loop(0, CHUNKS_PER_W // NBUF, kv_group, 0)


def _sc_mesh():
    return plsc.VectorSubcoreMesh(
        core_axis_name="c", subcore_axis_name="s",
        num_cores=NC, num_subcores=NS)


@functools.lru_cache(maxsize=1)
def _make_sc_gather():
    return functools.partial(
        pl.kernel,
        out_type=(
            jax.ShapeDtypeStruct((NROWS, EW), jnp.float32),
            jax.ShapeDtypeStruct((NROWS, EW), jnp.float32),
            jax.ShapeDtypeStruct((NROWS, EW), jnp.float32),
            jax.ShapeDtypeStruct((NROWS,), jnp.int32),
        ),
        mesh=_sc_mesh(),
        scratch_types=[
            pltpu.VMEM((CHUNKS_PER_W, CHUNK), jnp.int32),
            pltpu.VMEM((CHUNKS_PER_W, CHUNK), jnp.int32),
            pltpu.VMEM((CHUNKS_PER_W, CHUNK), jnp.int32),
            pltpu.VMEM((CHUNK,), jnp.int32),
            pltpu.VMEM((ROWS_PER_W,), jnp.int32),
            pltpu.VMEM((NBUF, CHUNK, EW), jnp.float32),
            pltpu.VMEM((NBUF, CHUNK, EW), jnp.float32),
            pltpu.SemaphoreType.DMA((NBUF,)),
            pltpu.SemaphoreType.DMA((NBUF,)),
        ],
        compiler_params=pltpu.CompilerParams(needs_layout_passes=False),
    )(_sc_gather_body)


def _sc_gather(*args):
    return _make_sc_gather()(*args)


GA = 8


def _attn_body(sq_ref, sk_ref, sv_ref, cq_ref, ck_ref, cv_ref,
               ext_ref, cbo_ref):
    dt = (((1,), (1,)), ((), ()))
    dk = (((0,), (0,)), ((), ()))
    cq = cq_ref[0]
    ck = ck_ref[0]
    cv = cv_ref[0]
    colid = lax.broadcasted_iota(jnp.int32, (BLK, EW), 1)
    qs = [sq_ref[g] for g in range(GA)]
    ks = [sk_ref[g] for g in range(GA)]
    vs = [sv_ref[g] for g in range(GA)]
    sts = [lax.dot_general(ks[g], qs[g], dt,
                           preferred_element_type=jnp.float32)
           for g in range(GA)]
    scrs = [lax.dot_general(ck, qs[g], dt,
                            preferred_element_type=jnp.float32)
            for g in range(GA)]
    tcols = [lax.dot_general(ks[g], cq, dt,
                             preferred_element_type=jnp.float32)
             for g in range(GA)]
    tcs = lax.dot_general(ck, cq, dt, preferred_element_type=jnp.float32)
    ets, ecs, dens, lses = [], [], [], []
    for g in range(GA):
        mr = jnp.maximum(jnp.max(sts[g], axis=0, keepdims=True), scrs[g])
        ets.append(jnp.exp(sts[g] - mr))
        ecs.append(jnp.exp(scrs[g] - mr))
        denr = jnp.sum(ets[g], axis=0, keepdims=True) + ecs[g]
        dens.append(denr)
        lses.append(mr + jnp.log(denr))
    pvs = [lax.dot_general(ets[g], vs[g], dk,
                           preferred_element_type=jnp.float32)
           for g in range(GA)]
    for g in range(GA):
        ec_c = jnp.transpose(ecs[g])
        den_c = jnp.transpose(dens[g])
        lse_c = jnp.transpose(lses[g])
        bo = (pvs[g] + ec_c * cv) / den_c
        ext_ref[g] = jnp.where(colid == DIM, lse_c, bo)
    e2s = []
    for g in range(GA):
        m2 = jnp.maximum(jnp.max(tcols[g], axis=0, keepdims=True), tcs)
        e2s.append((jnp.exp(tcols[g] - m2), jnp.exp(tcs - m2)))
    for g in range(GA):
        e2, e2c = e2s[g]
        den2 = jnp.sum(e2, axis=0, keepdims=True) + e2c
        cbo_ref[g] = (lax.dot_general(
            e2, vs[g], dk, preferred_element_type=jnp.float32)
            + e2c * cv) / den2


def _attention(sq, sk, sv, cls_q, cls_k, cls_v, interpret=False):
    nblk_total = NH * BS * NBLK
    blk_spec = pl.BlockSpec((GA, BLK, EW), lambda i: (i, 0, 0))
    cls_spec = pl.BlockSpec(
        (1, 1, EW), lambda i: ((i * GA) // NBLK % BS, 0, 0))
    return pl.pallas_call(
        _attn_body,
        grid=(nblk_total // GA,),
        in_specs=[blk_spec, blk_spec, blk_spec, cls_spec, cls_spec, cls_spec],
        out_specs=[
            pl.BlockSpec((GA, BLK, EW), lambda i: (i, 0, 0)),
            pl.BlockSpec((GA, 1, EW), lambda i: (i, 0, 0)),
        ],
        out_shape=[
            jax.ShapeDtypeStruct((nblk_total, BLK, EW), jnp.float32),
            jax.ShapeDtypeStruct((nblk_total, 1, EW), jnp.float32),
        ],
        interpret=interpret,
    )(sq, sk, sv, cls_q, cls_k, cls_v)


def _sc_merge_body(ext, gidx, out, i0_v, i1_v, r0, r1, ob, sem):
    w = _worker_id()

    def chunk(c, _):
        pltpu.sync_copy(gidx.at[0, w, c], i0_v)
        pltpu.sync_copy(gidx.at[1, w, c], i1_v)
        cp0 = pltpu.async_copy(ext.at[i0_v], r0, sem)
        cp1 = pltpu.async_copy(ext.at[i1_v], r1, sem)
        cp0.wait()
        cp1.wait()
        for r in range(BLK):
            la = r0[r, pl.ds(DIM, 16)]
            lb = r1[r, pl.ds(DIM, 16)]
            wv = 1.0 / (1.0 + jnp.exp(lb - la))
            ws = wv[0]
            for cg in range(4):
                a = r0[r, pl.ds(cg * 16, 16)]
                b = r1[r, pl.ds(cg * 16, 16)]
                ob[r, pl.ds(cg * 16, 16)] = b + ws * (a - b)
        off = pl.multiple_of((w * SEQ + c * BLK), BLK)
        pltpu.sync_copy(ob, out.at[pl.ds(off, BLK)])
        return 0

    lax.fori_loop(0, NBLK, chunk, 0)


@functools.lru_cache(maxsize=1)
def _make_sc_merge():
    return functools.partial(
        pl.kernel,
        out_type=jax.ShapeDtypeStruct((NTAB, EW), jnp.float32),
        mesh=_sc_mesh(),
        scratch_types=[
            pltpu.VMEM((BLK,), jnp.int32),
            pltpu.VMEM((BLK,), jnp.int32),
            pltpu.VMEM((BLK, EW), jnp.float32),
            pltpu.VMEM((BLK, EW), jnp.float32),
            pltpu.VMEM((BLK, EW), jnp.float32),
            pltpu.SemaphoreType.DMA,
        ],
    )(_sc_merge_body)


def _sc_merge(*args):
    return _make_sc_merge()(*args)


def kernel(queries, keys, values, alpha, beta):
    cls_q = queries[:, :1]
    cls_k = keys[:, :1]
    cls_v = values[:, :1]
    q = queries[:, 1:]
    k = keys[:, 1:]
    v = values[:, 1:]

    q_sg = lax.stop_gradient(q)
    k_sg = lax.stop_gradient(k)
    q_norm = jnp.linalg.norm(q_sg, axis=-1, keepdims=True)
    k_norm = jnp.linalg.norm(k_sg, axis=-1, keepdims=True)
    MQ = jnp.max(q_norm, axis=1, keepdims=True)
    MK = jnp.max(k_norm, axis=1, keepdims=True)
    q_ext = jnp.sqrt(jnp.maximum(MQ ** 2 + MK ** 2 - q_norm ** 2, 0.0))
    k_ext = jnp.sqrt(jnp.maximum(MQ ** 2 + MK ** 2 - k_norm ** 2, 0.0))
    Queries = jnp.concatenate([q_sg, q_ext, jnp.zeros_like(q_ext)], axis=-1)
    Keys = jnp.concatenate([k_sg, jnp.zeros_like(k_ext), k_ext], axis=-1)
    a = lax.stop_gradient(alpha)
    b = lax.stop_gradient(beta)
    q_hashed = jnp.transpose(Queries @ a + b, (2, 0, 1))
    k_hashed = jnp.transpose(Keys @ a + b, (2, 0, 1))
    q_positions = jnp.argsort(q_hashed, axis=-1).astype(jnp.int32)
    k_positions = jnp.argsort(k_hashed, axis=-1).astype(jnp.int32)

    boff = (jnp.arange(BS, dtype=jnp.int32) * SEQ)[None, :, None]
    q_gidx = (q_positions + boff).reshape(NW, CHUNKS_PER_W, CHUNK)
    k_gidx = (k_positions + boff).reshape(NW, CHUNKS_PER_W, CHUNK)
    rowpar = (jnp.arange(NH * BS, dtype=jnp.int32) % 2).reshape(
        NH, BS)[:, :, None] * SEQ
    rtidx = (q_positions + rowpar).reshape(NW, CHUNKS_PER_W, CHUNK)

    pad = ((0, 0), (0, EW - DIM))
    qt = jnp.pad(q.reshape(NTAB, DIM), pad)
    kt = jnp.pad(k.reshape(NTAB, DIM), pad)
    vt = jnp.pad(v.reshape(NTAB, DIM), pad)
    sq, sk, sv, rev = _sc_gather(qt, kt, vt, q_gidx, k_gidx, rtidx)

    pad3 = ((0, 0), (0, 0), (0, EW - DIM))
    nblk_total = NH * BS * NBLK
    ext, cbo = _attention(
        sq.reshape(nblk_total, BLK, EW),
        sk.reshape(nblk_total, BLK, EW),
        sv.reshape(nblk_total, BLK, EW),
        jnp.pad(cls_q, pad3), jnp.pad(cls_k, pad3), jnp.pad(cls_v, pad3))

    gidx = rev.reshape(NH, BS, NBLK, CHUNK)

    merged = _sc_merge(ext.reshape(NROWS, EW), gidx)
    out_body = merged[:, :DIM].reshape(BS, SEQ, DIM)
    cls_out = cbo.reshape(NH, BS, NBLK, EW)[..., :DIM].mean(
        axis=(0, 2))[:, None, :]
    return jnp.concatenate([cls_out, out_body], axis=1)


# speedup vs baseline: 7.9558x; 1.0146x over previous
import functools
import jax
import jax.numpy as jnp
from jax import lax
from jax.experimental import pallas as pl
from jax.experimental.pallas import tpu as pltpu
from jax.experimental.pallas import tpu_sc as plsc

NH = 2
BS = 32
SEQ = 4096
DIM = 64
BLK = 128
NBLK = SEQ // BLK
NROWS = NH * BS * SEQ
NTAB = BS * SEQ
NC, NS = 2, 16
NW = NC * NS
CHUNK = 128
ROWS_PER_W = NROWS // NW
CHUNKS_PER_W = ROWS_PER_W // CHUNK
NBUF = 2
EW = 128


def _worker_id():
    return lax.axis_index("s") * NC + lax.axis_index("c")


def _sc_gather_q_body(qt, qidx, rtidx, sq, rev,
                      qi_v, rt_v, ramp_v, rev_v, buf_a, sems_a):
    w = _worker_id()
    pltpu.sync_copy(qidx.at[w], qi_v)
    pltpu.sync_copy(rtidx.at[w], rt_v)
    base = w * ROWS_PER_W

    lane = lax.iota(jnp.int32, 16)
    for s in range(8):
        ramp_v[pl.ds(s * 16, 16)] = lane + (s * 16)

    def q_group(it, _):
        jo = it * NBUF
        cps = []
        for b in range(NBUF):
            cps.append(pltpu.async_copy(
                qt.at[qi_v.at[jo + b]], buf_a.at[b], sems_a.at[b]))
        for b in range(NBUF):
            cbase = base + (jo + b) * CHUNK
            for s in range(8):
                tgt = rt_v[jo + b, pl.ds(s * 16, 16)]
                vals = ramp_v[pl.ds(s * 16, 16)] + cbase
                plsc.store_scatter(rev_v, [tgt], vals)
        for b in range(NBUF):
            cps[b].wait()
            off = pl.multiple_of(base + (jo + b) * CHUNK, CHUNK)
            pltpu.sync_copy(buf_a.at[b], sq.at[pl.ds(off, CHUNK)])
        return 0

    lax.fori_loop(0, CHUNKS_PER_W // NBUF, q_group, 0)
    pltpu.sync_copy(rev_v, rev.at[pl.ds(base, ROWS_PER_W)])


def _sc_gather_kv_body(kt, vt, kidx, sk, sv,
                       ki_v, buf_a, buf_b, sems_a, sems_b):
    w = _worker_id()
    pltpu.sync_copy(kidx.at[w], ki_v)
    base = w * ROWS_PER_W

    def kv_group(it, _):
        jo = it * NBUF
        kcps, vcps = [], []
        for b in range(NBUF):
            kcps.append(pltpu.async_copy(
                kt.at[ki_v.at[jo + b]], buf_a.at[b], sems_a.at[b]))
            vcps.append(pltpu.async_copy(
                vt.at[ki_v.at[jo + b]], buf_b.at[b], sems_b.at[b]))
        for b in range(NBUF):
            off = pl.multiple_of(base + (jo + b) * CHUNK, CHUNK)
            kcps[b].wait()
            pltpu.sync_copy(buf_a.at[b], sk.at[pl.ds(off, CHUNK)])
            vcps[b].wait()
            pltpu.sync_copy(buf_b.at[b], sv.at[pl.ds(off, CHUNK)])
        return 0

    lax.fori_loop(0, CHUNKS_PER_W // NBUF, kv_group, 0)


def _sc_mesh():
    return plsc.VectorSubcoreMesh(
        core_axis_name="c", subcore_axis_name="s",
        num_cores=NC, num_subcores=NS)


@functools.lru_cache(maxsize=1)
def _make_sc_gather_q():
    return functools.partial(
        pl.kernel,
        out_type=(
            jax.ShapeDtypeStruct((NROWS, EW), jnp.float32),
            jax.ShapeDtypeStruct((NROWS,), jnp.int32),
        ),
        mesh=_sc_mesh(),
        scratch_types=[
            pltpu.VMEM((CHUNKS_PER_W, CHUNK), jnp.int32),
            pltpu.VMEM((CHUNKS_PER_W, CHUNK), jnp.int32),
            pltpu.VMEM((CHUNK,), jnp.int32),
            pltpu.VMEM((ROWS_PER_W,), jnp.int32),
            pltpu.VMEM((NBUF, CHUNK, EW), jnp.float32),
            pltpu.SemaphoreType.DMA((NBUF,)),
        ],
        compiler_params=pltpu.CompilerParams(needs_layout_passes=False),
    )(_sc_gather_q_body)


@functools.lru_cache(maxsize=1)
def _make_sc_gather_kv():
    return functools.partial(
        pl.kernel,
        out_type=(
            jax.ShapeDtypeStruct((NROWS, EW), jnp.float32),
            jax.ShapeDtypeStruct((NROWS, EW), jnp.float32),
        ),
        mesh=_sc_mesh(),
        scratch_types=[
            pltpu.VMEM((CHUNKS_PER_W, CHUNK), jnp.int32),
            pltpu.VMEM((NBUF, CHUNK, EW), jnp.float32),
            pltpu.VMEM((NBUF, CHUNK, EW), jnp.float32),
            pltpu.SemaphoreType.DMA((NBUF,)),
            pltpu.SemaphoreType.DMA((NBUF,)),
        ],
    )(_sc_gather_kv_body)


def _sc_gather_q(*args):
    return _make_sc_gather_q()(*args)


def _sc_gather_kv(*args):
    return _make_sc_gather_kv()(*args)


GA = 8


def _attn_body(sq_ref, sk_ref, sv_ref, cq_ref, ck_ref, cv_ref,
               ext_ref, cbo_ref):
    dt = (((1,), (1,)), ((), ()))
    dk = (((0,), (0,)), ((), ()))
    cq = cq_ref[0]
    ck = ck_ref[0]
    cv = cv_ref[0]
    colid = lax.broadcasted_iota(jnp.int32, (BLK, EW), 1)
    qs = [sq_ref[g] for g in range(GA)]
    ks = [sk_ref[g] for g in range(GA)]
    vs = [sv_ref[g] for g in range(GA)]
    sts = [lax.dot_general(ks[g], qs[g], dt,
                           preferred_element_type=jnp.float32)
           for g in range(GA)]
    scrs = [lax.dot_general(ck, qs[g], dt,
                            preferred_element_type=jnp.float32)
            for g in range(GA)]
    tcols = [lax.dot_general(ks[g], cq, dt,
                             preferred_element_type=jnp.float32)
             for g in range(GA)]
    tcs = lax.dot_general(ck, cq, dt, preferred_element_type=jnp.float32)
    ets, ecs, dens, lses = [], [], [], []
    for g in range(GA):
        mr = jnp.maximum(jnp.max(sts[g], axis=0, keepdims=True), scrs[g])
        ets.append(jnp.exp(sts[g] - mr))
        ecs.append(jnp.exp(scrs[g] - mr))
        denr = jnp.sum(ets[g], axis=0, keepdims=True) + ecs[g]
        dens.append(denr)
        lses.append(mr + jnp.log(denr))
    pvs = [lax.dot_general(ets[g], vs[g], dk,
                           preferred_element_type=jnp.float32)
           for g in range(GA)]
    for g in range(GA):
        ec_c = jnp.transpose(ecs[g])
        den_c = jnp.transpose(dens[g])
        lse_c = jnp.transpose(lses[g])
        bo = (pvs[g] + ec_c * cv) / den_c
        ext_ref[g] = jnp.where(colid == DIM, lse_c, bo)
    e2s = []
    for g in range(GA):
        m2 = jnp.maximum(jnp.max(tcols[g], axis=0, keepdims=True), tcs)
        e2s.append((jnp.exp(tcols[g] - m2), jnp.exp(tcs - m2)))
    for g in range(GA):
        e2, e2c = e2s[g]
        den2 = jnp.sum(e2, axis=0, keepdims=True) + e2c
        cbo_ref[g] = (lax.dot_general(
            e2, vs[g], dk, preferred_element_type=jnp.float32)
            + e2c * cv) / den2


def _attention(sq, sk, sv, cls_q, cls_k, cls_v, interpret=False):
    nblk_total = NH * BS * NBLK
    blk_spec = pl.BlockSpec((GA, BLK, EW), lambda i: (i, 0, 0))
    cls_spec = pl.BlockSpec(
        (1, 1, EW), lambda i: ((i * GA) // NBLK % BS, 0, 0))
    return pl.pallas_call(
        _attn_body,
        grid=(nblk_total // GA,),
        in_specs=[blk_spec, blk_spec, blk_spec, cls_spec, cls_spec, cls_spec],
        out_specs=[
            pl.BlockSpec((GA, BLK, EW), lambda i: (i, 0, 0)),
            pl.BlockSpec((GA, 1, EW), lambda i: (i, 0, 0)),
        ],
        out_shape=[
            jax.ShapeDtypeStruct((nblk_total, BLK, EW), jnp.float32),
            jax.ShapeDtypeStruct((nblk_total, 1, EW), jnp.float32),
        ],
        interpret=interpret,
    )(sq, sk, sv, cls_q, cls_k, cls_v)


def _sc_merge_body(ext, gidx, out, i0_v, i1_v, r0, r1, ob, sem):
    w = _worker_id()

    def chunk(c, _):
        pltpu.sync_copy(gidx.at[0, w, c], i0_v)
        pltpu.sync_copy(gidx.at[1, w, c], i1_v)
        cp0 = pltpu.async_copy(ext.at[i0_v], r0, sem)
        cp1 = pltpu.async_copy(ext.at[i1_v], r1, sem)
        cp0.wait()
        cp1.wait()
        for r in range(BLK):
            la = r0[r, pl.ds(DIM, 16)]
            lb = r1[r, pl.ds(DIM, 16)]
            wv = 1.0 / (1.0 + jnp.exp(lb - la))
            ws = wv[0]
            for cg in range(4):
                a = r0[r, pl.ds(cg * 16, 16)]
                b = r1[r, pl.ds(cg * 16, 16)]
                ob[r, pl.ds(cg * 16, 16)] = b + ws * (a - b)
        off = pl.multiple_of((w * SEQ + c * BLK), BLK)
        pltpu.sync_copy(ob, out.at[pl.ds(off, BLK)])
        return 0

    lax.fori_loop(0, NBLK, chunk, 0)


@functools.lru_cache(maxsize=1)
def _make_sc_merge():
    return functools.partial(
        pl.kernel,
        out_type=jax.ShapeDtypeStruct((NTAB, EW), jnp.float32),
        mesh=_sc_mesh(),
        scratch_types=[
            pltpu.VMEM((BLK,), jnp.int32),
            pltpu.VMEM((BLK,), jnp.int32),
            pltpu.VMEM((BLK, EW), jnp.float32),
            pltpu.VMEM((BLK, EW), jnp.float32),
            pltpu.VMEM((BLK, EW), jnp.float32),
            pltpu.SemaphoreType.DMA,
        ],
    )(_sc_merge_body)


def _sc_merge(*args):
    return _make_sc_merge()(*args)


def kernel(queries, keys, values, alpha, beta):
    cls_q = queries[:, :1]
    cls_k = keys[:, :1]
    cls_v = values[:, :1]
    q = queries[:, 1:]
    k = keys[:, 1:]
    v = values[:, 1:]

    q_sg = lax.stop_gradient(q)
    k_sg = lax.stop_gradient(k)
    q_norm = jnp.linalg.norm(q_sg, axis=-1, keepdims=True)
    k_norm = jnp.linalg.norm(k_sg, axis=-1, keepdims=True)
    MQ = jnp.max(q_norm, axis=1, keepdims=True)
    MK = jnp.max(k_norm, axis=1, keepdims=True)
    q_ext = jnp.sqrt(jnp.maximum(MQ ** 2 + MK ** 2 - q_norm ** 2, 0.0))
    k_ext = jnp.sqrt(jnp.maximum(MQ ** 2 + MK ** 2 - k_norm ** 2, 0.0))
    Queries = jnp.concatenate([q_sg, q_ext, jnp.zeros_like(q_ext)], axis=-1)
    Keys = jnp.concatenate([k_sg, jnp.zeros_like(k_ext), k_ext], axis=-1)
    a = lax.stop_gradient(alpha)
    b = lax.stop_gradient(beta)
    q_hashed = jnp.transpose(Queries @ a + b, (2, 0, 1))
    k_hashed = jnp.transpose(Keys @ a + b, (2, 0, 1))
    q_positions = jnp.argsort(q_hashed, axis=-1).astype(jnp.int32)
    k_positions = jnp.argsort(k_hashed, axis=-1).astype(jnp.int32)

    boff = (jnp.arange(BS, dtype=jnp.int32) * SEQ)[None, :, None]
    q_gidx = (q_positions + boff).reshape(NW, CHUNKS_PER_W, CHUNK)
    k_gidx = (k_positions + boff).reshape(NW, CHUNKS_PER_W, CHUNK)
    rowpar = (jnp.arange(NH * BS, dtype=jnp.int32) % 2).reshape(
        NH, BS)[:, :, None] * SEQ
    rtidx = (q_positions + rowpar).reshape(NW, CHUNKS_PER_W, CHUNK)

    pad = ((0, 0), (0, EW - DIM))
    qt = jnp.pad(q.reshape(NTAB, DIM), pad)
    kt = jnp.pad(k.reshape(NTAB, DIM), pad)
    vt = jnp.pad(v.reshape(NTAB, DIM), pad)
    sq, rev = _sc_gather_q(qt, q_gidx, rtidx)
    sk, sv = _sc_gather_kv(kt, vt, k_gidx)

    pad3 = ((0, 0), (0, 0), (0, EW - DIM))
    nblk_total = NH * BS * NBLK
    ext, cbo = _attention(
        sq.reshape(nblk_total, BLK, EW),
        sk.reshape(nblk_total, BLK, EW),
        sv.reshape(nblk_total, BLK, EW),
        jnp.pad(cls_q, pad3), jnp.pad(cls_k, pad3), jnp.pad(cls_v, pad3))

    gidx = rev.reshape(NH, BS, NBLK, CHUNK)

    merged = _sc_merge(ext.reshape(NROWS, EW), gidx)
    out_body = merged[:, :DIM].reshape(BS, SEQ, DIM)
    cls_out = cbo.reshape(NH, BS, NBLK, EW)[..., :DIM].mean(
        axis=(0, 2))[:, None, :]
    return jnp.concatenate([cls_out, out_body], axis=1)


# speedup vs baseline: 7.9573x; 1.0002x over previous
import functools
import jax
import jax.numpy as jnp
from jax import lax
from jax.experimental import pallas as pl
from jax.experimental.pallas import tpu as pltpu
from jax.experimental.pallas import tpu_sc as plsc

NH = 2
BS = 32
SEQ = 4096
DIM = 64
BLK = 128
NBLK = SEQ // BLK
NROWS = NH * BS * SEQ
NTAB = BS * SEQ
NC, NS = 2, 16
NW = NC * NS
CHUNK = 128
ROWS_PER_W = NROWS // NW
CHUNKS_PER_W = ROWS_PER_W // CHUNK
NBUF = 2
NBUFQ = 4
EW = 128


def _worker_id():
    return lax.axis_index("s") * NC + lax.axis_index("c")


NROW_W = ROWS_PER_W // SEQ
NCH_ROW = SEQ // CHUNK


def _chunk_slice(idx_v, j):
    return idx_v.at[lax.div(j, NCH_ROW), pl.ds(lax.rem(j, NCH_ROW) * CHUNK,
                                               CHUNK)]


def _sc_gather_q_body(qt, qidx, rtidx, sq, rev,
                      qi_v, rt_v, ramp_v, rev_v, buf_a, sems_a):
    w = _worker_id()
    pltpu.sync_copy(qidx.at[pl.ds(w * NROW_W, NROW_W)], qi_v)
    pltpu.sync_copy(rtidx.at[pl.ds(w * NROW_W, NROW_W)], rt_v)
    base = w * ROWS_PER_W

    lane = lax.iota(jnp.int32, 16)
    for s in range(8):
        ramp_v[pl.ds(s * 16, 16)] = lane + (s * 16)

    def q_group(it, _):
        jo = it * NBUFQ
        cps = []
        for b in range(NBUFQ):
            cps.append(pltpu.async_copy(
                qt.at[_chunk_slice(qi_v, jo + b)], buf_a.at[b],
                sems_a.at[b]))
        for b in range(NBUFQ):
            j = jo + b
            r = lax.div(j, NCH_ROW)
            c = lax.rem(j, NCH_ROW) * CHUNK
            cbase = base + j * CHUNK
            for s in range(8):
                tgt = rt_v[r, pl.ds(c + s * 16, 16)]
                vals = ramp_v[pl.ds(s * 16, 16)] + cbase
                ri = lax.shift_right_logical(tgt, 12)
                ci = lax.bitwise_and(tgt, SEQ - 1)
                plsc.store_scatter(rev_v, [ri, ci], vals)
        for b in range(NBUFQ):
            cps[b].wait()
            off = pl.multiple_of(base + (jo + b) * CHUNK, CHUNK)
            pltpu.sync_copy(buf_a.at[b], sq.at[pl.ds(off, CHUNK)])
        return 0

    lax.fori_loop(0, CHUNKS_PER_W // NBUFQ, q_group, 0)
    pltpu.sync_copy(rev_v, rev.at[pl.ds(w * NROW_W, NROW_W)])


def _sc_gather_kv_body(kt, vt, kidx, sk, sv,
                       ki_v, buf_a, buf_b, sems_a, sems_b):
    w = _worker_id()
    pltpu.sync_copy(kidx.at[pl.ds(w * NROW_W, NROW_W)], ki_v)
    base = w * ROWS_PER_W

    def kv_group(it, _):
        jo = it * NBUF
        kcps, vcps = [], []
        for b in range(NBUF):
            sl = _chunk_slice(ki_v, jo + b)
            kcps.append(pltpu.async_copy(
                kt.at[sl], buf_a.at[b], sems_a.at[b]))
            vcps.append(pltpu.async_copy(
                vt.at[sl], buf_b.at[b], sems_b.at[b]))
        for b in range(NBUF):
            off = pl.multiple_of(base + (jo + b) * CHUNK, CHUNK)
            kcps[b].wait()
            pltpu.sync_copy(buf_a.at[b], sk.at[pl.ds(off, CHUNK)])
            vcps[b].wait()
            pltpu.sync_copy(buf_b.at[b], sv.at[pl.ds(off, CHUNK)])
        return 0

    lax.fori_loop(0, CHUNKS_PER_W // NBUF, kv_group, 0)


def _sc_mesh():
    return plsc.VectorSubcoreMesh(
        core_axis_name="c", subcore_axis_name="s",
        num_cores=NC, num_subcores=NS)


@functools.lru_cache(maxsize=1)
def _make_sc_gather_q():
    return functools.partial(
        pl.kernel,
        out_type=(
            jax.ShapeDtypeStruct((NROWS, EW), jnp.float32),
            jax.ShapeDtypeStruct((NH * BS, SEQ), jnp.int32),
        ),
        mesh=_sc_mesh(),
        scratch_types=[
            pltpu.VMEM((NROW_W, SEQ), jnp.int32),
            pltpu.VMEM((NROW_W, SEQ), jnp.int32),
            pltpu.VMEM((CHUNK,), jnp.int32),
            pltpu.VMEM((NROW_W, SEQ), jnp.int32),
            pltpu.VMEM((NBUFQ, CHUNK, EW), jnp.float32),
            pltpu.SemaphoreType.DMA((NBUFQ,)),
        ],
        compiler_params=pltpu.CompilerParams(needs_layout_passes=False),
    )(_sc_gather_q_body)


@functools.lru_cache(maxsize=1)
def _make_sc_gather_kv():
    return functools.partial(
        pl.kernel,
        out_type=(
            jax.ShapeDtypeStruct((NROWS, EW), jnp.float32),
            jax.ShapeDtypeStruct((NROWS, EW), jnp.float32),
        ),
        mesh=_sc_mesh(),
        scratch_types=[
            pltpu.VMEM((NROW_W, SEQ), jnp.int32),
            pltpu.VMEM((NBUF, CHUNK, EW), jnp.float32),
            pltpu.VMEM((NBUF, CHUNK, EW), jnp.float32),
            pltpu.SemaphoreType.DMA((NBUF,)),
            pltpu.SemaphoreType.DMA((NBUF,)),
        ],
    )(_sc_gather_kv_body)


def _sc_gather_q(*args):
    return _make_sc_gather_q()(*args)


def _sc_gather_kv(*args):
    return _make_sc_gather_kv()(*args)


GA = 8


def _attn_body(sq_ref, sk_ref, sv_ref, cq_ref, ck_ref, cv_ref,
               ext_ref, cbo_ref):
    dt = (((1,), (1,)), ((), ()))
    dk = (((0,), (0,)), ((), ()))
    cq = cq_ref[0]
    ck = ck_ref[0]
    cv = cv_ref[0]
    colid = lax.broadcasted_iota(jnp.int32, (BLK, EW), 1)
    qs = [sq_ref[g] for g in range(GA)]
    ks = [sk_ref[g] for g in range(GA)]
    vs = [sv_ref[g] for g in range(GA)]
    sts = [lax.dot_general(ks[g], qs[g], dt,
                           preferred_element_type=jnp.float32)
           for g in range(GA)]
    scrs = [lax.dot_general(ck, qs[g], dt,
                            preferred_element_type=jnp.float32)
            for g in range(GA)]
    tcols = [lax.dot_general(ks[g], cq, dt,
                             preferred_element_type=jnp.float32)
             for g in range(GA)]
    tcs = lax.dot_general(ck, cq, dt, preferred_element_type=jnp.float32)
    ets, ecs, dens, lses = [], [], [], []
    for g in range(GA):
        mr = jnp.maximum(jnp.max(sts[g], axis=0, keepdims=True), scrs[g])
        ets.append(jnp.exp(sts[g] - mr))
        ecs.append(jnp.exp(scrs[g] - mr))
        denr = jnp.sum(ets[g], axis=0, keepdims=True) + ecs[g]
        dens.append(denr)
        lses.append(mr + jnp.log(denr))
    pvs = [lax.dot_general(ets[g], vs[g], dk,
                           preferred_element_type=jnp.float32)
           for g in range(GA)]
    for g in range(GA):
        ec_c = jnp.transpose(ecs[g])
        den_c = jnp.transpose(dens[g])
        lse_c = jnp.transpose(lses[g])
        bo = (pvs[g] + ec_c * cv) / den_c
        ext_ref[g] = jnp.where(colid == DIM, lse_c, bo)
    e2s = []
    for g in range(GA):
        m2 = jnp.maximum(jnp.max(tcols[g], axis=0, keepdims=True), tcs)
        e2s.append((jnp.exp(tcols[g] - m2), jnp.exp(tcs - m2)))
    for g in range(GA):
        e2, e2c = e2s[g]
        den2 = jnp.sum(e2, axis=0, keepdims=True) + e2c
        cbo_ref[g] = (lax.dot_general(
            e2, vs[g], dk, preferred_element_type=jnp.float32)
            + e2c * cv) / den2


def _attention(sq, sk, sv, cls_q, cls_k, cls_v, interpret=False):
    nblk_total = NH * BS * NBLK
    blk_spec = pl.BlockSpec((GA, BLK, EW), lambda i: (i, 0, 0))
    cls_spec = pl.BlockSpec(
        (1, 1, EW), lambda i: ((i * GA) // NBLK % BS, 0, 0))
    return pl.pallas_call(
        _attn_body,
        grid=(nblk_total // GA,),
        in_specs=[blk_spec, blk_spec, blk_spec, cls_spec, cls_spec, cls_spec],
        out_specs=[
            pl.BlockSpec((GA, BLK, EW), lambda i: (i, 0, 0)),
            pl.BlockSpec((GA, 1, EW), lambda i: (i, 0, 0)),
        ],
        out_shape=[
            jax.ShapeDtypeStruct((nblk_total, BLK, EW), jnp.float32),
            jax.ShapeDtypeStruct((nblk_total, 1, EW), jnp.float32),
        ],
        interpret=interpret,
    )(sq, sk, sv, cls_q, cls_k, cls_v)


def _sc_merge_body(ext, gidx, out, i0_v, i1_v, r0, r1, ob, sem):
    w = _worker_id()

    def chunk(c, _):
        pltpu.sync_copy(gidx.at[w, pl.ds(c * CHUNK, CHUNK)], i0_v)
        pltpu.sync_copy(gidx.at[BS + w, pl.ds(c * CHUNK, CHUNK)], i1_v)
        cp0 = pltpu.async_copy(ext.at[i0_v], r0, sem)
        cp1 = pltpu.async_copy(ext.at[i1_v], r1, sem)
        cp0.wait()
        cp1.wait()
        for r in range(BLK):
            la = r0[r, pl.ds(DIM, 16)]
            lb = r1[r, pl.ds(DIM, 16)]
            wv = 1.0 / (1.0 + jnp.exp(lb - la))
            ws = wv[0]
            for cg in range(4):
                a = r0[r, pl.ds(cg * 16, 16)]
                b = r1[r, pl.ds(cg * 16, 16)]
                ob[r, pl.ds(cg * 16, 16)] = b + ws * (a - b)
        off = pl.multiple_of((w * SEQ + c * BLK), BLK)
        pltpu.sync_copy(ob, out.at[pl.ds(off, BLK)])
        return 0

    lax.fori_loop(0, NBLK, chunk, 0)


@functools.lru_cache(maxsize=1)
def _make_sc_merge():
    return functools.partial(
        pl.kernel,
        out_type=jax.ShapeDtypeStruct((NTAB, EW), jnp.float32),
        mesh=_sc_mesh(),
        scratch_types=[
            pltpu.VMEM((BLK,), jnp.int32),
            pltpu.VMEM((BLK,), jnp.int32),
            pltpu.VMEM((BLK, EW), jnp.float32),
            pltpu.VMEM((BLK, EW), jnp.float32),
            pltpu.VMEM((BLK, EW), jnp.float32),
            pltpu.SemaphoreType.DMA,
        ],
    )(_sc_merge_body)


def _sc_merge(*args):
    return _make_sc_merge()(*args)


def kernel(queries, keys, values, alpha, beta):
    cls_q = queries[:, :1]
    cls_k = keys[:, :1]
    cls_v = values[:, :1]
    q = queries[:, 1:]
    k = keys[:, 1:]
    v = values[:, 1:]

    q_sg = lax.stop_gradient(q)
    k_sg = lax.stop_gradient(k)
    q_norm = jnp.linalg.norm(q_sg, axis=-1, keepdims=True)
    k_norm = jnp.linalg.norm(k_sg, axis=-1, keepdims=True)
    MQ = jnp.max(q_norm, axis=1, keepdims=True)
    MK = jnp.max(k_norm, axis=1, keepdims=True)
    q_ext = jnp.sqrt(jnp.maximum(MQ ** 2 + MK ** 2 - q_norm ** 2, 0.0))
    k_ext = jnp.sqrt(jnp.maximum(MQ ** 2 + MK ** 2 - k_norm ** 2, 0.0))
    Queries = jnp.concatenate([q_sg, q_ext, jnp.zeros_like(q_ext)], axis=-1)
    Keys = jnp.concatenate([k_sg, jnp.zeros_like(k_ext), k_ext], axis=-1)
    a = lax.stop_gradient(alpha)
    b = lax.stop_gradient(beta)
    q_hashed = jnp.transpose(Queries @ a + b, (2, 0, 1))
    k_hashed = jnp.transpose(Keys @ a + b, (2, 0, 1))
    q_positions = jnp.argsort(q_hashed, axis=-1).astype(jnp.int32)
    k_positions = jnp.argsort(k_hashed, axis=-1).astype(jnp.int32)

    boff = (jnp.arange(BS, dtype=jnp.int32) * SEQ)[None, :, None]
    q_gidx = (q_positions + boff).reshape(NH * BS, SEQ)
    k_gidx = (k_positions + boff).reshape(NH * BS, SEQ)
    rowpar = (jnp.arange(NH * BS, dtype=jnp.int32) % 2)[:, None] * SEQ
    rtidx = q_positions.reshape(NH * BS, SEQ) + rowpar

    pad = ((0, 0), (0, EW - DIM))
    qt = jnp.pad(q.reshape(NTAB, DIM), pad)
    kt = jnp.pad(k.reshape(NTAB, DIM), pad)
    vt = jnp.pad(v.reshape(NTAB, DIM), pad)
    sq, rev = _sc_gather_q(qt, q_gidx, rtidx)
    sk, sv = _sc_gather_kv(kt, vt, k_gidx)

    pad3 = ((0, 0), (0, 0), (0, EW - DIM))
    nblk_total = NH * BS * NBLK
    ext, cbo = _attention(
        sq.reshape(nblk_total, BLK, EW),
        sk.reshape(nblk_total, BLK, EW),
        sv.reshape(nblk_total, BLK, EW),
        jnp.pad(cls_q, pad3), jnp.pad(cls_k, pad3), jnp.pad(cls_v, pad3))

    merged = _sc_merge(ext.reshape(NROWS, EW), rev)
    out_body = merged[:, :DIM].reshape(BS, SEQ, DIM)
    cls_out = cbo.reshape(NH, BS, NBLK, EW)[..., :DIM].mean(
        axis=(0, 2))[:, None, :]
    return jnp.concatenate([cls_out, out_body], axis=1)


# speedup vs baseline: 7.9722x; 1.0019x over previous
import functools
import jax
import jax.numpy as jnp
from jax import lax
from jax.experimental import pallas as pl
from jax.experimental.pallas import tpu as pltpu
from jax.experimental.pallas import tpu_sc as plsc

NH = 2
BS = 32
SEQ = 4096
DIM = 64
BLK = 128
NBLK = SEQ // BLK
NROWS = NH * BS * SEQ
NTAB = BS * SEQ
NC, NS = 2, 16
NW = NC * NS
CHUNK = 128
ROWS_PER_W = NROWS // NW
CHUNKS_PER_W = ROWS_PER_W // CHUNK
NBUF = 2
NBUFQ = 4
EW = 128


def _worker_id():
    return lax.axis_index("s") * NC + lax.axis_index("c")


NROW_W = ROWS_PER_W // SEQ
NCH_ROW = SEQ // CHUNK


def _chunk_slice(idx_v, j):
    return idx_v.at[lax.div(j, NCH_ROW), pl.ds(lax.rem(j, NCH_ROW) * CHUNK,
                                               CHUNK)]


def _sc_gather_q_body(qt, qidx, rtidx, sq, rev,
                      qi_v, rt_v, ramp_v, rev_v, buf_a, sems_a):
    w = _worker_id()
    pltpu.sync_copy(qidx.at[pl.ds(w * NROW_W, NROW_W)], qi_v)
    pltpu.sync_copy(rtidx.at[pl.ds(w * NROW_W, NROW_W)], rt_v)
    base = w * ROWS_PER_W

    lane = lax.iota(jnp.int32, 16)
    for s in range(8):
        ramp_v[pl.ds(s * 16, 16)] = lane + (s * 16)

    def q_group(it, _):
        jo = it * NBUFQ
        cps = []
        for b in range(NBUFQ):
            cps.append(pltpu.async_copy(
                qt.at[_chunk_slice(qi_v, jo + b)], buf_a.at[b],
                sems_a.at[b]))
        for b in range(NBUFQ):
            j = jo + b
            r = lax.div(j, NCH_ROW)
            c = lax.rem(j, NCH_ROW) * CHUNK
            cbase = base + j * CHUNK
            for s in range(8):
                tgt = rt_v[r, pl.ds(c + s * 16, 16)]
                vals = ramp_v[pl.ds(s * 16, 16)] + cbase
                ri = lax.shift_right_logical(tgt, 12)
                ci = lax.bitwise_and(tgt, SEQ - 1)
                plsc.store_scatter(rev_v, [ri, ci], vals)
        for b in range(NBUFQ):
            cps[b].wait()
            off = pl.multiple_of(base + (jo + b) * CHUNK, CHUNK)
            pltpu.sync_copy(buf_a.at[b], sq.at[pl.ds(off, CHUNK)])
        return 0

    lax.fori_loop(0, CHUNKS_PER_W // NBUFQ, q_group, 0)
    pltpu.sync_copy(rev_v, rev.at[pl.ds(w * NROW_W, NROW_W)])


def _sc_gather_kv_body(kt, vt, kidx, sk, sv,
                       ki_v, buf_a, buf_b, sems_a, sems_b):
    w = _worker_id()
    pltpu.sync_copy(kidx.at[pl.ds(w * NROW_W, NROW_W)], ki_v)
    base = w * ROWS_PER_W

    def kv_group(it, _):
        jo = it * NBUF
        kcps, vcps = [], []
        for b in range(NBUF):
            sl = _chunk_slice(ki_v, jo + b)
            kcps.append(pltpu.async_copy(
                kt.at[sl], buf_a.at[b], sems_a.at[b]))
            vcps.append(pltpu.async_copy(
                vt.at[sl], buf_b.at[b], sems_b.at[b]))
        for b in range(NBUF):
            off = pl.multiple_of(base + (jo + b) * CHUNK, CHUNK)
            kcps[b].wait()
            pltpu.sync_copy(buf_a.at[b], sk.at[pl.ds(off, CHUNK)])
            vcps[b].wait()
            pltpu.sync_copy(buf_b.at[b], sv.at[pl.ds(off, CHUNK)])
        return 0

    lax.fori_loop(0, CHUNKS_PER_W // NBUF, kv_group, 0)


def _sc_mesh():
    return plsc.VectorSubcoreMesh(
        core_axis_name="c", subcore_axis_name="s",
        num_cores=NC, num_subcores=NS)


@functools.lru_cache(maxsize=1)
def _make_sc_gather_q():
    return functools.partial(
        pl.kernel,
        out_type=(
            jax.ShapeDtypeStruct((NROWS, EW), jnp.float32),
            jax.ShapeDtypeStruct((NH * BS, SEQ), jnp.int32),
        ),
        mesh=_sc_mesh(),
        scratch_types=[
            pltpu.VMEM((NROW_W, SEQ), jnp.int32),
            pltpu.VMEM((NROW_W, SEQ), jnp.int32),
            pltpu.VMEM((CHUNK,), jnp.int32),
            pltpu.VMEM((NROW_W, SEQ), jnp.int32),
            pltpu.VMEM((NBUFQ, CHUNK, EW), jnp.float32),
            pltpu.SemaphoreType.DMA((NBUFQ,)),
        ],
        compiler_params=pltpu.CompilerParams(needs_layout_passes=False),
    )(_sc_gather_q_body)


@functools.lru_cache(maxsize=1)
def _make_sc_gather_kv():
    return functools.partial(
        pl.kernel,
        out_type=(
            jax.ShapeDtypeStruct((NROWS, EW), jnp.float32),
            jax.ShapeDtypeStruct((NROWS, EW), jnp.float32),
        ),
        mesh=_sc_mesh(),
        scratch_types=[
            pltpu.VMEM((NROW_W, SEQ), jnp.int32),
            pltpu.VMEM((NBUF, CHUNK, EW), jnp.float32),
            pltpu.VMEM((NBUF, CHUNK, EW), jnp.float32),
            pltpu.SemaphoreType.DMA((NBUF,)),
            pltpu.SemaphoreType.DMA((NBUF,)),
        ],
    )(_sc_gather_kv_body)


def _sc_gather_q(*args):
    return _make_sc_gather_q()(*args)


def _sc_gather_kv(*args):
    return _make_sc_gather_kv()(*args)


GA = 8


def _attn_body(sq_ref, sk_ref, sv_ref, cq_ref, ck_ref, cv_ref,
               ext_ref, cbo_ref):
    dt = (((1,), (1,)), ((), ()))
    dk = (((0,), (0,)), ((), ()))
    cq = cq_ref[0]
    ck = ck_ref[0]
    cv = cv_ref[0]
    colid = lax.broadcasted_iota(jnp.int32, (BLK, EW), 1)
    qs = [sq_ref[g] for g in range(GA)]
    ks = [sk_ref[g] for g in range(GA)]
    vs = [sv_ref[g] for g in range(GA)]
    sts = [lax.dot_general(ks[g], qs[g], dt,
                           preferred_element_type=jnp.float32)
           for g in range(GA)]
    scrs = [lax.dot_general(ck, qs[g], dt,
                            preferred_element_type=jnp.float32)
            for g in range(GA)]
    tcols = [lax.dot_general(ks[g], cq, dt,
                             preferred_element_type=jnp.float32)
             for g in range(GA)]
    tcs = lax.dot_general(ck, cq, dt, preferred_element_type=jnp.float32)
    ets, ecs, dens, lses = [], [], [], []
    for g in range(GA):
        mr = jnp.maximum(jnp.max(sts[g], axis=0, keepdims=True), scrs[g])
        ets.append(jnp.exp(sts[g] - mr))
        ecs.append(jnp.exp(scrs[g] - mr))
        denr = jnp.sum(ets[g], axis=0, keepdims=True) + ecs[g]
        dens.append(denr)
        lses.append(mr + jnp.log(denr))
    pvs = [lax.dot_general(ets[g], vs[g], dk,
                           preferred_element_type=jnp.float32)
           for g in range(GA)]
    for g in range(GA):
        ec_c = jnp.transpose(ecs[g])
        den_c = jnp.transpose(dens[g])
        lse_c = jnp.transpose(lses[g])
        bo = (pvs[g] + ec_c * cv) / den_c
        ext_ref[g] = jnp.where(colid == DIM, lse_c, bo)
    e2s = []
    for g in range(GA):
        m2 = jnp.maximum(jnp.max(tcols[g], axis=0, keepdims=True), tcs)
        e2s.append((jnp.exp(tcols[g] - m2), jnp.exp(tcs - m2)))
    for g in range(GA):
        e2, e2c = e2s[g]
        den2 = jnp.sum(e2, axis=0, keepdims=True) + e2c
        cbo_ref[g] = (lax.dot_general(
            e2, vs[g], dk, preferred_element_type=jnp.float32)
            + e2c * cv) / den2


def _attention(sq, sk, sv, cls_q, cls_k, cls_v, interpret=False):
    nblk_total = NH * BS * NBLK
    blk_spec = pl.BlockSpec((GA, BLK, EW), lambda i: (i, 0, 0))
    cls_spec = pl.BlockSpec(
        (1, 1, EW), lambda i: ((i * GA) // NBLK % BS, 0, 0))
    return pl.pallas_call(
        _attn_body,
        grid=(nblk_total // GA,),
        in_specs=[blk_spec, blk_spec, blk_spec, cls_spec, cls_spec, cls_spec],
        out_specs=[
            pl.BlockSpec((GA, BLK, EW), lambda i: (i, 0, 0)),
            pl.BlockSpec((GA, 1, EW), lambda i: (i, 0, 0)),
        ],
        out_shape=[
            jax.ShapeDtypeStruct((nblk_total, BLK, EW), jnp.float32),
            jax.ShapeDtypeStruct((nblk_total, 1, EW), jnp.float32),
        ],
        interpret=interpret,
    )(sq, sk, sv, cls_q, cls_k, cls_v)


def _sc_merge_body(ext, gidx, out, i0_v, i1_v, r0, r1, ob, sem):
    w = _worker_id()

    def chunk(c, _):
        pltpu.sync_copy(gidx.at[w, pl.ds(c * CHUNK, CHUNK)], i0_v)
        pltpu.sync_copy(gidx.at[BS + w, pl.ds(c * CHUNK, CHUNK)], i1_v)
        cp0 = pltpu.async_copy(ext.at[i0_v], r0, sem)
        cp1 = pltpu.async_copy(ext.at[i1_v], r1, sem)
        cp0.wait()
        cp1.wait()
        for r in range(BLK):
            la = r0[r, pl.ds(DIM, 16)]
            lb = r1[r, pl.ds(DIM, 16)]
            wv = 1.0 / (1.0 + jnp.exp(lb - la))
            ws = wv[0]
            for cg in range(4):
                a = r0[r, pl.ds(cg * 16, 16)]
                b = r1[r, pl.ds(cg * 16, 16)]
                ob[r, pl.ds(cg * 16, 16)] = b + ws * (a - b)
        off = pl.multiple_of((w * SEQ + c * BLK), BLK)
        pltpu.sync_copy(ob, out.at[pl.ds(off, BLK)])
        return 0

    lax.fori_loop(0, NBLK, chunk, 0)


@functools.lru_cache(maxsize=1)
def _make_sc_merge():
    return functools.partial(
        pl.kernel,
        out_type=jax.ShapeDtypeStruct((NTAB, EW), jnp.float32),
        mesh=_sc_mesh(),
        scratch_types=[
            pltpu.VMEM((BLK,), jnp.int32),
            pltpu.VMEM((BLK,), jnp.int32),
            pltpu.VMEM((BLK, EW), jnp.float32),
            pltpu.VMEM((BLK, EW), jnp.float32),
            pltpu.VMEM((BLK, EW), jnp.float32),
            pltpu.SemaphoreType.DMA,
        ],
    )(_sc_merge_body)


def _sc_merge(*args):
    return _make_sc_merge()(*args)


def kernel(queries, keys, values, alpha, beta):
    cls_q = queries[:, :1]
    cls_k = keys[:, :1]
    cls_v = values[:, :1]
    q = queries[:, 1:]
    k = keys[:, 1:]
    v = values[:, 1:]

    q_sg = lax.stop_gradient(q)
    k_sg = lax.stop_gradient(k)
    q_norm = jnp.linalg.norm(q_sg, axis=-1, keepdims=True)
    k_norm = jnp.linalg.norm(k_sg, axis=-1, keepdims=True)
    MQ = jnp.max(q_norm, axis=1, keepdims=True)
    MK = jnp.max(k_norm, axis=1, keepdims=True)
    q_ext = jnp.sqrt(jnp.maximum(MQ ** 2 + MK ** 2 - q_norm ** 2, 0.0))
    k_ext = jnp.sqrt(jnp.maximum(MQ ** 2 + MK ** 2 - k_norm ** 2, 0.0))
    Queries = jnp.concatenate([q_sg, q_ext, jnp.zeros_like(q_ext)], axis=-1)
    Keys = jnp.concatenate([k_sg, jnp.zeros_like(k_ext), k_ext], axis=-1)
    a = lax.stop_gradient(alpha)
    b = lax.stop_gradient(beta)
    q_hashed = jnp.transpose(Queries @ a + b, (2, 0, 1))
    k_hashed = jnp.transpose(Keys @ a + b, (2, 0, 1))
    q_positions = jnp.argsort(q_hashed, axis=-1).astype(jnp.int32)

    boff = (jnp.arange(BS, dtype=jnp.int32) * SEQ)[None, :, None]
    q_gidx = (q_positions + boff).reshape(NH * BS, SEQ)
    rowpar = (jnp.arange(NH * BS, dtype=jnp.int32) % 2)[:, None] * SEQ
    rtidx = q_positions.reshape(NH * BS, SEQ) + rowpar

    pad = ((0, 0), (0, EW - DIM))
    qt = jnp.pad(q.reshape(NTAB, DIM), pad)
    kt = jnp.pad(k.reshape(NTAB, DIM), pad)
    vt = jnp.pad(v.reshape(NTAB, DIM), pad)
    sq, rev = _sc_gather_q(qt, q_gidx, rtidx)
    k_positions = jnp.argsort(k_hashed, axis=-1).astype(jnp.int32)
    k_gidx = (k_positions + boff).reshape(NH * BS, SEQ)
    sk, sv = _sc_gather_kv(kt, vt, k_gidx)

    pad3 = ((0, 0), (0, 0), (0, EW - DIM))
    nblk_total = NH * BS * NBLK
    ext, cbo = _attention(
        sq.reshape(nblk_total, BLK, EW),
        sk.reshape(nblk_total, BLK, EW),
        sv.reshape(nblk_total, BLK, EW),
        jnp.pad(cls_q, pad3), jnp.pad(cls_k, pad3), jnp.pad(cls_v, pad3))

    merged = _sc_merge(ext.reshape(NROWS, EW), rev)
    out_body = merged[:, :DIM].reshape(BS, SEQ, DIM)
    cls_out = cbo.reshape(NH, BS, NBLK, EW)[..., :DIM].mean(
        axis=(0, 2))[:, None, :]
    return jnp.concatenate([cls_out, out_body], axis=1)


# speedup vs baseline: 7.9863x; 1.0018x over previous
import functools
import jax
import jax.numpy as jnp
from jax import lax
from jax.experimental import pallas as pl
from jax.experimental.pallas import tpu as pltpu
from jax.experimental.pallas import tpu_sc as plsc

NH = 2
BS = 32
SEQ = 4096
DIM = 64
BLK = 128
NBLK = SEQ // BLK
NROWS = NH * BS * SEQ
NTAB = BS * SEQ
NC, NS = 2, 16
NW = NC * NS
CHUNK = 128
ROWS_PER_W = NROWS // NW
CHUNKS_PER_W = ROWS_PER_W // CHUNK
NBUF = 2
NBUFQ = 4
EW = 128


def _worker_id():
    return lax.axis_index("s") * NC + lax.axis_index("c")


NROW_W = ROWS_PER_W // SEQ
NCH_ROW = SEQ // CHUNK


def _chunk_slice(idx_v, j):
    return idx_v.at[lax.div(j, NCH_ROW), pl.ds(lax.rem(j, NCH_ROW) * CHUNK,
                                               CHUNK)]


def _sc_gather_q_body(qt, qidx, rtidx, sq, rev,
                      qi_v, rt_v, ramp_v, rev_v, buf_a, sems_a):
    w = _worker_id()
    pltpu.sync_copy(qidx.at[pl.ds(w * NROW_W, NROW_W)], qi_v)
    pltpu.sync_copy(rtidx.at[pl.ds(w * NROW_W, NROW_W)], rt_v)
    base = w * ROWS_PER_W

    lane = lax.iota(jnp.int32, 16)
    for s in range(8):
        ramp_v[pl.ds(s * 16, 16)] = lane + (s * 16)

    def q_group(it, _):
        jo = it * NBUFQ
        cps = []
        for b in range(NBUFQ):
            cps.append(pltpu.async_copy(
                qt.at[_chunk_slice(qi_v, jo + b)], buf_a.at[b],
                sems_a.at[b]))
        for b in range(NBUFQ):
            j = jo + b
            r = lax.div(j, NCH_ROW)
            c = lax.rem(j, NCH_ROW) * CHUNK
            cbase = base + j * CHUNK
            for s in range(8):
                tgt = rt_v[r, pl.ds(c + s * 16, 16)]
                vals = ramp_v[pl.ds(s * 16, 16)] + cbase
                ri = lax.shift_right_logical(tgt, 12)
                ci = lax.bitwise_and(tgt, SEQ - 1)
                plsc.store_scatter(rev_v, [ri, ci], vals)
        for b in range(NBUFQ):
            cps[b].wait()
            off = pl.multiple_of(base + (jo + b) * CHUNK, CHUNK)
            pltpu.sync_copy(buf_a.at[b], sq.at[pl.ds(off, CHUNK)])
        return 0

    lax.fori_loop(0, CHUNKS_PER_W // NBUFQ, q_group, 0)
    pltpu.sync_copy(rev_v, rev.at[pl.ds(w * NROW_W, NROW_W)])


def _sc_gather_kv_body(kt, vt, kidx, sk, sv,
                       ki_v, buf_a, buf_b, sems_a, sems_b):
    w = _worker_id()
    pltpu.sync_copy(kidx.at[pl.ds(w * NROW_W, NROW_W)], ki_v)
    base = w * ROWS_PER_W

    def kv_group(it, _):
        jo = it * NBUF
        kcps, vcps = [], []
        for b in range(NBUF):
            sl = _chunk_slice(ki_v, jo + b)
            kcps.append(pltpu.async_copy(
                kt.at[sl], buf_a.at[b], sems_a.at[b]))
            vcps.append(pltpu.async_copy(
                vt.at[sl], buf_b.at[b], sems_b.at[b]))
        for b in range(NBUF):
            off = pl.multiple_of(base + (jo + b) * CHUNK, CHUNK)
            kcps[b].wait()
            pltpu.sync_copy(buf_a.at[b], sk.at[pl.ds(off, CHUNK)])
            vcps[b].wait()
            pltpu.sync_copy(buf_b.at[b], sv.at[pl.ds(off, CHUNK)])
        return 0

    lax.fori_loop(0, CHUNKS_PER_W // NBUF, kv_group, 0)


def _sc_mesh():
    return plsc.VectorSubcoreMesh(
        core_axis_name="c", subcore_axis_name="s",
        num_cores=NC, num_subcores=NS)


@functools.lru_cache(maxsize=1)
def _make_sc_gather_q():
    return functools.partial(
        pl.kernel,
        out_type=(
            jax.ShapeDtypeStruct((NROWS, EW), jnp.float32),
            jax.ShapeDtypeStruct((NH * BS, SEQ), jnp.int32),
        ),
        mesh=_sc_mesh(),
        scratch_types=[
            pltpu.VMEM((NROW_W, SEQ), jnp.int32),
            pltpu.VMEM((NROW_W, SEQ), jnp.int32),
            pltpu.VMEM((CHUNK,), jnp.int32),
            pltpu.VMEM((NROW_W, SEQ), jnp.int32),
            pltpu.VMEM((NBUFQ, CHUNK, EW), jnp.float32),
            pltpu.SemaphoreType.DMA((NBUFQ,)),
        ],
        compiler_params=pltpu.CompilerParams(needs_layout_passes=False),
    )(_sc_gather_q_body)


@functools.lru_cache(maxsize=1)
def _make_sc_gather_kv():
    return functools.partial(
        pl.kernel,
        out_type=(
            jax.ShapeDtypeStruct((NROWS, EW), jnp.float32),
            jax.ShapeDtypeStruct((NROWS, EW), jnp.float32),
        ),
        mesh=_sc_mesh(),
        scratch_types=[
            pltpu.VMEM((NROW_W, SEQ), jnp.int32),
            pltpu.VMEM((NBUF, CHUNK, EW), jnp.float32),
            pltpu.VMEM((NBUF, CHUNK, EW), jnp.float32),
            pltpu.SemaphoreType.DMA((NBUF,)),
            pltpu.SemaphoreType.DMA((NBUF,)),
        ],
    )(_sc_gather_kv_body)


def _sc_gather_q(*args):
    return _make_sc_gather_q()(*args)


def _sc_gather_kv(*args):
    return _make_sc_gather_kv()(*args)


GA = 8


def _attn_body(sq_ref, sk_ref, sv_ref, cq_ref, ck_ref, cv_ref,
               ext_ref, cbo_ref):
    dt = (((1,), (1,)), ((), ()))
    dk = (((0,), (0,)), ((), ()))
    cq = cq_ref[0]
    ck = ck_ref[0]
    cv = cv_ref[0]
    colid = lax.broadcasted_iota(jnp.int32, (BLK, EW), 1)
    qs = [sq_ref[g] for g in range(GA)]
    ks = [sk_ref[g] for g in range(GA)]
    vs = [sv_ref[g] for g in range(GA)]
    sts = [lax.dot_general(ks[g], qs[g], dt,
                           preferred_element_type=jnp.float32)
           for g in range(GA)]
    scrs = [lax.dot_general(ck, qs[g], dt,
                            preferred_element_type=jnp.float32)
            for g in range(GA)]
    tcols = [lax.dot_general(ks[g], cq, dt,
                             preferred_element_type=jnp.float32)
             for g in range(GA)]
    tcs = lax.dot_general(ck, cq, dt, preferred_element_type=jnp.float32)
    ets, ecs, dens, lses = [], [], [], []
    for g in range(GA):
        mr = jnp.maximum(jnp.max(sts[g], axis=0, keepdims=True), scrs[g])
        ets.append(jnp.exp(sts[g] - mr))
        ecs.append(jnp.exp(scrs[g] - mr))
        denr = jnp.sum(ets[g], axis=0, keepdims=True) + ecs[g]
        dens.append(denr)
        lses.append(mr + jnp.log(denr))
    pvs = [lax.dot_general(ets[g], vs[g], dk,
                           preferred_element_type=jnp.float32)
           for g in range(GA)]
    for g in range(GA):
        ec_c = jnp.transpose(ecs[g])
        den_c = jnp.transpose(dens[g])
        lse_c = jnp.transpose(lses[g])
        bo = (pvs[g] + ec_c * cv) / den_c
        ext_ref[g] = jnp.where(colid == DIM, lse_c, bo)
    e2s = []
    for g in range(GA):
        m2 = jnp.maximum(jnp.max(tcols[g], axis=0, keepdims=True), tcs)
        e2s.append((jnp.exp(tcols[g] - m2), jnp.exp(tcs - m2)))
    for g in range(GA):
        e2, e2c = e2s[g]
        den2 = jnp.sum(e2, axis=0, keepdims=True) + e2c
        cbo_ref[g] = (lax.dot_general(
            e2, vs[g], dk, preferred_element_type=jnp.float32)
            + e2c * cv) / den2


def _attention(sq, sk, sv, cls_q, cls_k, cls_v, interpret=False):
    nblk_total = NH * BS * NBLK
    blk_spec = pl.BlockSpec((GA, BLK, EW), lambda i: (i, 0, 0))
    cls_spec = pl.BlockSpec(
        (1, 1, EW), lambda i: ((i * GA) // NBLK % BS, 0, 0))
    return pl.pallas_call(
        _attn_body,
        grid=(nblk_total // GA,),
        in_specs=[blk_spec, blk_spec, blk_spec, cls_spec, cls_spec, cls_spec],
        out_specs=[
            pl.BlockSpec((GA, BLK, EW), lambda i: (i, 0, 0)),
            pl.BlockSpec((GA, 1, EW), lambda i: (i, 0, 0)),
        ],
        out_shape=[
            jax.ShapeDtypeStruct((nblk_total, BLK, EW), jnp.float32),
            jax.ShapeDtypeStruct((nblk_total, 1, EW), jnp.float32),
        ],
        interpret=interpret,
    )(sq, sk, sv, cls_q, cls_k, cls_v)


def _sc_merge_body(ext, gidx, out, i0_v, i1_v, r0, r1, ob, sem):
    w = _worker_id()

    def pair_start(c, s):
        pltpu.sync_copy(gidx.at[w, pl.ds(c * CHUNK, CHUNK)], i0_v.at[s])
        pltpu.sync_copy(gidx.at[BS + w, pl.ds(c * CHUNK, CHUNK)], i1_v.at[s])
        return (pltpu.async_copy(ext.at[i0_v.at[s]], r0.at[s], sem),
                pltpu.async_copy(ext.at[i1_v.at[s]], r1.at[s], sem))

    def compute(c, s):
        for r in range(BLK):
            la = r0[s, r, pl.ds(DIM, 16)]
            lb = r1[s, r, pl.ds(DIM, 16)]
            wv = 1.0 / (1.0 + jnp.exp(lb - la))
            ws = wv[0]
            for cg in range(4):
                a = r0[s, r, pl.ds(cg * 16, 16)]
                b = r1[s, r, pl.ds(cg * 16, 16)]
                ob[r, pl.ds(cg * 16, 16)] = b + ws * (a - b)
        off = pl.multiple_of(w * SEQ + c * BLK, BLK)
        pltpu.sync_copy(ob, out.at[pl.ds(off, BLK)])

    def pair(it, _):
        c0 = it * 2
        cps0 = pair_start(c0, 0)
        cps1 = pair_start(c0 + 1, 1)
        for cp in cps0 + cps1:
            cp.wait()
        compute(c0, 0)
        compute(c0 + 1, 1)
        return 0

    lax.fori_loop(0, NBLK // 2, pair, 0)


@functools.lru_cache(maxsize=1)
def _make_sc_merge():
    return functools.partial(
        pl.kernel,
        out_type=jax.ShapeDtypeStruct((NTAB, DIM), jnp.float32),
        mesh=_sc_mesh(),
        scratch_types=[
            pltpu.VMEM((2, BLK), jnp.int32),
            pltpu.VMEM((2, BLK), jnp.int32),
            pltpu.VMEM((2, BLK, EW), jnp.float32),
            pltpu.VMEM((2, BLK, EW), jnp.float32),
            pltpu.VMEM((BLK, DIM), jnp.float32),
            pltpu.SemaphoreType.DMA,
        ],
    )(_sc_merge_body)


def _sc_merge(*args):
    return _make_sc_merge()(*args)


def kernel(queries, keys, values, alpha, beta):
    cls_q = queries[:, :1]
    cls_k = keys[:, :1]
    cls_v = values[:, :1]
    q = queries[:, 1:]
    k = keys[:, 1:]
    v = values[:, 1:]

    q_sg = lax.stop_gradient(q)
    k_sg = lax.stop_gradient(k)
    q_norm = jnp.linalg.norm(q_sg, axis=-1, keepdims=True)
    k_norm = jnp.linalg.norm(k_sg, axis=-1, keepdims=True)
    MQ = jnp.max(q_norm, axis=1, keepdims=True)
    MK = jnp.max(k_norm, axis=1, keepdims=True)
    q_ext = jnp.sqrt(jnp.maximum(MQ ** 2 + MK ** 2 - q_norm ** 2, 0.0))
    k_ext = jnp.sqrt(jnp.maximum(MQ ** 2 + MK ** 2 - k_norm ** 2, 0.0))
    Queries = jnp.concatenate([q_sg, q_ext, jnp.zeros_like(q_ext)], axis=-1)
    Keys = jnp.concatenate([k_sg, jnp.zeros_like(k_ext), k_ext], axis=-1)
    a = lax.stop_gradient(alpha)
    b = lax.stop_gradient(beta)
    q_hashed = jnp.transpose(Queries @ a + b, (2, 0, 1))
    k_hashed = jnp.transpose(Keys @ a + b, (2, 0, 1))
    q_positions = jnp.argsort(q_hashed, axis=-1).astype(jnp.int32)

    boff = (jnp.arange(BS, dtype=jnp.int32) * SEQ)[None, :, None]
    q_gidx = (q_positions + boff).reshape(NH * BS, SEQ)
    rowpar = (jnp.arange(NH * BS, dtype=jnp.int32) % 2)[:, None] * SEQ
    rtidx = q_positions.reshape(NH * BS, SEQ) + rowpar

    pad = ((0, 0), (0, EW - DIM))
    qt = jnp.pad(q.reshape(NTAB, DIM), pad)
    kt = jnp.pad(k.reshape(NTAB, DIM), pad)
    vt = jnp.pad(v.reshape(NTAB, DIM), pad)
    sq, rev = _sc_gather_q(qt, q_gidx, rtidx)
    k_positions = jnp.argsort(k_hashed, axis=-1).astype(jnp.int32)
    k_gidx = (k_positions + boff).reshape(NH * BS, SEQ)
    sk, sv = _sc_gather_kv(kt, vt, k_gidx)

    pad3 = ((0, 0), (0, 0), (0, EW - DIM))
    nblk_total = NH * BS * NBLK
    ext, cbo = _attention(
        sq.reshape(nblk_total, BLK, EW),
        sk.reshape(nblk_total, BLK, EW),
        sv.reshape(nblk_total, BLK, EW),
        jnp.pad(cls_q, pad3), jnp.pad(cls_k, pad3), jnp.pad(cls_v, pad3))

    merged = _sc_merge(ext.reshape(NROWS, EW), rev)
    cls_out = cbo.reshape(NH, BS, NBLK, EW)[..., :DIM].mean(
        axis=(0, 2))[:, None, :]
    return jnp.concatenate([cls_out, merged.reshape(BS, SEQ, DIM)], axis=1)
